# Initial kernel scaffold; baseline (speedup 1.0000x reference)
import jax
import jax.numpy as jnp
from jax.experimental import pallas as pl


def kernel(x, tables, W, b, gamma, beta):
    raise NotImplementedError("write your pallas kernel here")


# speedup vs baseline: 7.0930x; 7.0930x over previous
import functools

import jax
import jax.numpy as jnp
from jax import lax
from jax.experimental import pallas as pl
from jax.experimental.pallas import tpu as pltpu
from jax.experimental.pallas import tpu_sc as plsc

_F = 26
_V = 100001
_E = 50
_EP = 128
_B = 16384
_TEP = _F * _EP
_OUT = 128
_EPS = 1e-5
_BLK = 512


def _sc_gather(tab3, x3):
    info = plsc.get_sparse_core_info()
    nc, ns = info.num_cores, info.num_subcores
    nw = nc * ns
    bpw = _B // nw
    nch = bpw // 128

    @functools.partial(
        pl.kernel,
        mesh=plsc.VectorSubcoreMesh(core_axis_name="c", subcore_axis_name="s"),
        out_type=jax.ShapeDtypeStruct((_F, _B, _EP), jnp.float32),
        scratch_types=[
            pltpu.VMEM((nch, 128), jnp.int32),
            pltpu.VMEM((bpw, _EP), jnp.float32),
            pltpu.SemaphoreType.DMA,
        ],
    )
    def gather_kernel(tab_hbm, x_hbm, out_hbm, idx_v, rows_v, sem):
        wid = lax.axis_index("s") * nc + lax.axis_index("c")
        base = wid * bpw

        def field_body(f, carry):
            pltpu.sync_copy(x_hbm.at[f, wid], idx_v)

            def gdesc(c):
                return pltpu.make_async_copy(
                    tab_hbm.at[f].at[idx_v.at[c]],
                    rows_v.at[pl.ds(c * 128, 128)],
                    sem,
                )

            for c in range(nch):
                gdesc(c).start()
            for c in range(nch):
                gdesc(c).wait()
            pltpu.sync_copy(rows_v, out_hbm.at[f, pl.ds(base, bpw), :])
            return carry

        lax.fori_loop(0, _F, field_body, 0)

    return gather_kernel(tab3, x3)


def _tc_dense(emb, W, b, gamma, beta):
    nb = _B // _BLK

    def body(emb_ref, w_ref, b_ref, g_ref, bt_ref, out_ref, y_buf, acc):
        p = pl.program_id(0)
        i = pl.program_id(1)

        @pl.when(p == 0)
        def _phase0():
            flat = jnp.concatenate(
                [emb_ref[f] for f in range(_F)], axis=1
            )
            y = (
                jnp.dot(flat, w_ref[...], preferred_element_type=jnp.float32)
                + b_ref[...]
            )
            y_buf[pl.ds(i * _BLK, _BLK), :] = y

            @pl.when(i == 0)
            def _init():
                acc[...] = jnp.zeros_like(acc)

            acc[0:1, :] = acc[0:1, :] + jnp.sum(y, axis=0, keepdims=True)
            acc[1:2, :] = acc[1:2, :] + jnp.sum(y * y, axis=0, keepdims=True)

        @pl.when(p == 1)
        def _phase1():
            mean = acc[0:1, :] * (1.0 / _B)
            var = acc[1:2, :] * (1.0 / _B) - mean * mean
            scale = g_ref[...] * lax.rsqrt(var + _EPS)
            shift = bt_ref[...] - mean * scale
            out_ref[...] = y_buf[pl.ds(i * _BLK, _BLK), :] * scale + shift

    return pl.pallas_call(
        body,
        grid=(2, nb),
        in_specs=[
            pl.BlockSpec((_F, _BLK, _EP), lambda p, i: (0, (1 - p) * i, 0)),
            pl.BlockSpec((_TEP, _OUT), lambda p, i: (0, 0)),
            pl.BlockSpec((1, _OUT), lambda p, i: (0, 0)),
            pl.BlockSpec((1, _OUT), lambda p, i: (0, 0)),
            pl.BlockSpec((1, _OUT), lambda p, i: (0, 0)),
        ],
        out_specs=pl.BlockSpec((_BLK, _OUT), lambda p, i: (i, 0)),
        out_shape=jax.ShapeDtypeStruct((_B, _OUT), jnp.float32),
        scratch_shapes=[
            pltpu.VMEM((_B, _OUT), jnp.float32),
            pltpu.VMEM((8, _OUT), jnp.float32),
        ],
    )(emb, W, b.reshape(1, _OUT), gamma.reshape(1, _OUT),
      beta.reshape(1, _OUT))


def kernel(x, tables, W, b, gamma, beta):
    x3 = x.reshape(_F, _B // _BLK, _BLK // 128, 128)
    tabp = jnp.pad(tables, ((0, 0), (0, 0), (0, _EP - _E)))
    Wp = jnp.pad(
        W.reshape(_F, _E, _OUT), ((0, 0), (0, _EP - _E), (0, 0))
    ).reshape(_TEP, _OUT)
    emb = _sc_gather(tabp, x3)
    return _tc_dense(emb, Wp, b, gamma, beta)


# speedup vs baseline: 9.0374x; 1.2741x over previous
import functools

import jax
import jax.numpy as jnp
from jax import lax
from jax.experimental import pallas as pl
from jax.experimental.pallas import tpu as pltpu
from jax.experimental.pallas import tpu_sc as plsc

_F = 26
_V = 100001
_E = 50
_B = 16384
_OUT = 128
_EPS = 1e-5
_BLK = 512
_VB = 2048
_VP = -(-_V // _VB) * _VB


def _tc_project(tabT, W3):
    def body(t_ref, w_ref, p_ref):
        p = lax.dot_general(
            t_ref[0],
            w_ref[0],
            (((0,), (0,)), ((), ())),
            preferred_element_type=jnp.float32,
        )
        p_ref[0] = p

    return pl.pallas_call(
        body,
        grid=(_F, _VP // _VB),
        in_specs=[
            pl.BlockSpec((1, _E, _VB), lambda f, v: (f, 0, v)),
            pl.BlockSpec((1, _E, _OUT), lambda f, v: (f, 0, 0)),
        ],
        out_specs=pl.BlockSpec((1, _VB, _OUT), lambda f, v: (f, v, 0)),
        out_shape=jax.ShapeDtypeStruct((_F, _VP, _OUT), jnp.float32),
    )(tabT, W3)


def _sc_gather(p_hbm_arr, x3):
    info = plsc.get_sparse_core_info()
    nc, ns = info.num_cores, info.num_subcores
    nw = nc * ns
    bpw = _B // nw
    nch = bpw // 128

    @functools.partial(
        pl.kernel,
        mesh=plsc.VectorSubcoreMesh(core_axis_name="c", subcore_axis_name="s"),
        out_type=jax.ShapeDtypeStruct((_F, _B, _OUT), jnp.float32),
        scratch_types=[
            pltpu.VMEM((nch, 128), jnp.int32),
            pltpu.VMEM((bpw, _OUT), jnp.float32),
            pltpu.SemaphoreType.DMA,
        ],
    )
    def gather_kernel(p_hbm, x_hbm, out_hbm, idx_v, rows_v, sem):
        wid = lax.axis_index("s") * nc + lax.axis_index("c")
        base = wid * bpw

        def field_body(f, carry):
            pltpu.sync_copy(x_hbm.at[f, wid], idx_v)

            def gdesc(c):
                return pltpu.make_async_copy(
                    p_hbm.at[f].at[idx_v.at[c]],
                    rows_v.at[pl.ds(c * 128, 128)],
                    sem,
                )

            for c in range(nch):
                gdesc(c).start()
            for c in range(nch):
                gdesc(c).wait()
            pltpu.sync_copy(rows_v, out_hbm.at[f, pl.ds(base, bpw), :])
            return carry

        lax.fori_loop(0, _F, field_body, 0)

    return gather_kernel(p_hbm_arr, x3)


def _tc_sum_bn(g, gamma, beta):
    nb = _B // _BLK

    def body(g_ref, gm_ref, bt_ref, out_ref, y_buf, acc):
        p = pl.program_id(0)
        i = pl.program_id(1)

        @pl.when(p == 0)
        def _phase0():
            y = jnp.sum(g_ref[...], axis=0)
            y_buf[pl.ds(i * _BLK, _BLK), :] = y

            @pl.when(i == 0)
            def _init():
                acc[...] = jnp.zeros_like(acc)

            acc[0:1, :] = acc[0:1, :] + jnp.sum(y, axis=0, keepdims=True)
            acc[1:2, :] = acc[1:2, :] + jnp.sum(y * y, axis=0, keepdims=True)

        @pl.when(p == 1)
        def _phase1():
            mean = acc[0:1, :] * (1.0 / _B)
            var = acc[1:2, :] * (1.0 / _B) - mean * mean
            scale = gm_ref[...] * lax.rsqrt(var + _EPS)
            shift = bt_ref[...] - mean * scale
            out_ref[...] = y_buf[pl.ds(i * _BLK, _BLK), :] * scale + shift

    return pl.pallas_call(
        body,
        grid=(2, nb),
        in_specs=[
            pl.BlockSpec((_F, _BLK, _OUT), lambda p, i: (0, (1 - p) * i, 0)),
            pl.BlockSpec((1, _OUT), lambda p, i: (0, 0)),
            pl.BlockSpec((1, _OUT), lambda p, i: (0, 0)),
        ],
        out_specs=pl.BlockSpec((_BLK, _OUT), lambda p, i: (i, 0)),
        out_shape=jax.ShapeDtypeStruct((_B, _OUT), jnp.float32),
        scratch_shapes=[
            pltpu.VMEM((_B, _OUT), jnp.float32),
            pltpu.VMEM((8, _OUT), jnp.float32),
        ],
    )(g, gamma.reshape(1, _OUT), beta.reshape(1, _OUT))


def kernel(x, tables, W, b, gamma, beta):
    del b
    tabT = jnp.transpose(tables, (0, 2, 1))
    W3 = W.reshape(_F, _E, _OUT)
    P = _tc_project(tabT, W3)
    x3 = x.reshape(_F, _B // _BLK, (_B // (_B // _BLK)) // 128, 128)
    g = _sc_gather(P, x3)
    return _tc_sum_bn(g, gamma, beta)


# speedup vs baseline: 11.6387x; 1.2878x over previous
import functools

import jax
import jax.numpy as jnp
from jax import lax
from jax.experimental import pallas as pl
from jax.experimental.pallas import tpu as pltpu
from jax.experimental.pallas import tpu_sc as plsc

_F = 26
_V = 100001
_E = 50
_B = 16384
_OUT = 128
_EPS = 1e-5
_BLK = 512
_VB = 4096
_VP = -(-_V // _VB) * _VB


def _tc_project(tabT, W3):
    def body(t_ref, w_ref, p_ref):
        p = lax.dot_general(
            t_ref[0],
            w_ref[0],
            (((0,), (0,)), ((), ())),
            preferred_element_type=jnp.float32,
        )
        p_ref[0] = p

    return pl.pallas_call(
        body,
        grid=(_F, _VP // _VB),
        in_specs=[
            pl.BlockSpec((1, _E, _VB), lambda f, v: (f, 0, v)),
            pl.BlockSpec((1, _E, _OUT), lambda f, v: (f, 0, 0)),
        ],
        out_specs=pl.BlockSpec((1, _VB, _OUT), lambda f, v: (f, v, 0)),
        out_shape=jax.ShapeDtypeStruct((_F, _VP, _OUT), jnp.float32),
    )(tabT, W3)


def _sc_gather(p_hbm_arr, x3):
    info = plsc.get_sparse_core_info()
    nc, ns = info.num_cores, info.num_subcores
    nw = nc * ns
    bpw = _B // nw
    nch = bpw // 128

    @functools.partial(
        pl.kernel,
        mesh=plsc.VectorSubcoreMesh(core_axis_name="c", subcore_axis_name="s"),
        out_type=jax.ShapeDtypeStruct((_F, _B, _OUT), jnp.float32),
        scratch_types=[
            pltpu.VMEM((nch, 128), jnp.int32),
            pltpu.VMEM((bpw, _OUT), jnp.float32),
            pltpu.SemaphoreType.DMA,
        ],
    )
    def gather_kernel(p_hbm, x_hbm, out_hbm, idx_v, rows_v, sem):
        wid = lax.axis_index("s") * nc + lax.axis_index("c")
        base = wid * bpw

        def field_body(f, carry):
            pltpu.sync_copy(x_hbm.at[f, wid], idx_v)

            def gdesc(c):
                return pltpu.make_async_copy(
                    p_hbm.at[f].at[idx_v.at[c]],
                    rows_v.at[pl.ds(c * 128, 128)],
                    sem,
                )

            for c in range(nch):
                gdesc(c).start()
            for c in range(nch):
                gdesc(c).wait()
            pltpu.sync_copy(rows_v, out_hbm.at[f, pl.ds(base, bpw), :])
            return carry

        lax.fori_loop(0, _F, field_body, 0)

    return gather_kernel(p_hbm_arr, x3)


def _tc_sum_bn(g, gamma, beta):
    nb = _B // _BLK

    def body(g_ref, gm_ref, bt_ref, out_ref, y_buf, acc):
        p = pl.program_id(0)
        i = pl.program_id(1)

        @pl.when(p == 0)
        def _phase0():
            y = jnp.sum(g_ref[...], axis=0)
            y_buf[pl.ds(i * _BLK, _BLK), :] = y

            @pl.when(i == 0)
            def _init():
                acc[...] = jnp.zeros_like(acc)

            acc[0:1, :] = acc[0:1, :] + jnp.sum(y, axis=0, keepdims=True)
            acc[1:2, :] = acc[1:2, :] + jnp.sum(y * y, axis=0, keepdims=True)

        @pl.when(p == 1)
        def _phase1():
            mean = acc[0:1, :] * (1.0 / _B)
            var = acc[1:2, :] * (1.0 / _B) - mean * mean
            scale = gm_ref[...] * lax.rsqrt(var + _EPS)
            shift = bt_ref[...] - mean * scale
            out_ref[...] = y_buf[pl.ds(i * _BLK, _BLK), :] * scale + shift

    return pl.pallas_call(
        body,
        grid=(2, nb),
        in_specs=[
            pl.BlockSpec((_F, _BLK, _OUT), lambda p, i: (0, (1 - p) * i, 0)),
            pl.BlockSpec((1, _OUT), lambda p, i: (0, 0)),
            pl.BlockSpec((1, _OUT), lambda p, i: (0, 0)),
        ],
        out_specs=pl.BlockSpec((_BLK, _OUT), lambda p, i: (i, 0)),
        out_shape=jax.ShapeDtypeStruct((_B, _OUT), jnp.float32),
        scratch_shapes=[
            pltpu.VMEM((_B, _OUT), jnp.float32),
            pltpu.VMEM((8, _OUT), jnp.float32),
        ],
    )(g, gamma.reshape(1, _OUT), beta.reshape(1, _OUT))


def kernel(x, tables, W, b, gamma, beta):
    del b
    tabT = jnp.transpose(tables, (0, 2, 1))
    W3 = W.reshape(_F, _E, _OUT)
    P = _tc_project(tabT, W3)
    x3 = x.reshape(_F, _B // _BLK, (_B // (_B // _BLK)) // 128, 128)
    g = _sc_gather(P, x3)
    return _tc_sum_bn(g, gamma, beta)


# speedup vs baseline: 13.7578x; 1.1821x over previous
import functools

import jax
import jax.numpy as jnp
from jax import lax
from jax.experimental import pallas as pl
from jax.experimental.pallas import tpu as pltpu
from jax.experimental.pallas import tpu_sc as plsc

_F = 26
_V = 100001
_E = 50
_B = 16384
_OUT = 128
_EPS = 1e-5
_BLK = 512
_VB = 8192
_VP = -(-_V // _VB) * _VB


def _tc_project(tabT, W3):
    def body(t_ref, w_ref, p_ref):
        p = lax.dot_general(
            t_ref[0],
            w_ref[0],
            (((0,), (0,)), ((), ())),
            preferred_element_type=jnp.float32,
        )
        p_ref[0] = p

    return pl.pallas_call(
        body,
        grid=(_F, _VP // _VB),
        in_specs=[
            pl.BlockSpec((1, _E, _VB), lambda f, v: (f, 0, v)),
            pl.BlockSpec((1, _E, _OUT), lambda f, v: (f, 0, 0)),
        ],
        out_specs=pl.BlockSpec((1, _VB, _OUT), lambda f, v: (f, v, 0)),
        out_shape=jax.ShapeDtypeStruct((_F, _VP, _OUT), jnp.float32),
    )(tabT, W3)


def _sc_gather(p_hbm_arr, x3):
    info = plsc.get_sparse_core_info()
    nc, ns = info.num_cores, info.num_subcores
    nw = nc * ns
    bpw = _B // nw
    nch = bpw // 128

    @functools.partial(
        pl.kernel,
        mesh=plsc.VectorSubcoreMesh(core_axis_name="c", subcore_axis_name="s"),
        out_type=jax.ShapeDtypeStruct((_F, _B, _OUT), jnp.float32),
        scratch_types=[
            pltpu.VMEM((nch, 128), jnp.int32),
            pltpu.VMEM((bpw, _OUT), jnp.float32),
            pltpu.SemaphoreType.DMA,
        ],
    )
    def gather_kernel(p_hbm, x_hbm, out_hbm, idx_v, rows_v, sem):
        wid = lax.axis_index("s") * nc + lax.axis_index("c")
        base = wid * bpw

        def field_body(f, carry):
            pltpu.sync_copy(x_hbm.at[f, wid], idx_v)

            def gdesc(c):
                return pltpu.make_async_copy(
                    p_hbm.at[f].at[idx_v.at[c]],
                    rows_v.at[pl.ds(c * 128, 128)],
                    sem,
                )

            for c in range(nch):
                gdesc(c).start()
            for c in range(nch):
                gdesc(c).wait()
            pltpu.sync_copy(rows_v, out_hbm.at[f, pl.ds(base, bpw), :])
            return carry

        lax.fori_loop(0, _F, field_body, 0)

    return gather_kernel(p_hbm_arr, x3)


def _tc_sum_bn(g, gamma, beta):
    nb = _B // _BLK

    def body(g_ref, gm_ref, bt_ref, out_ref, y_buf, acc):
        p = pl.program_id(0)
        i = pl.program_id(1)

        @pl.when(p == 0)
        def _phase0():
            y = jnp.sum(g_ref[...], axis=0)
            y_buf[pl.ds(i * _BLK, _BLK), :] = y

            @pl.when(i == 0)
            def _init():
                acc[...] = jnp.zeros_like(acc)

            acc[0:1, :] = acc[0:1, :] + jnp.sum(y, axis=0, keepdims=True)
            acc[1:2, :] = acc[1:2, :] + jnp.sum(y * y, axis=0, keepdims=True)

        @pl.when(p == 1)
        def _phase1():
            mean = acc[0:1, :] * (1.0 / _B)
            var = acc[1:2, :] * (1.0 / _B) - mean * mean
            scale = gm_ref[...] * lax.rsqrt(var + _EPS)
            shift = bt_ref[...] - mean * scale
            out_ref[...] = y_buf[pl.ds(i * _BLK, _BLK), :] * scale + shift

    return pl.pallas_call(
        body,
        grid=(2, nb),
        in_specs=[
            pl.BlockSpec((_F, _BLK, _OUT), lambda p, i: (0, (1 - p) * i, 0)),
            pl.BlockSpec((1, _OUT), lambda p, i: (0, 0)),
            pl.BlockSpec((1, _OUT), lambda p, i: (0, 0)),
        ],
        out_specs=pl.BlockSpec((_BLK, _OUT), lambda p, i: (i, 0)),
        out_shape=jax.ShapeDtypeStruct((_B, _OUT), jnp.float32),
        scratch_shapes=[
            pltpu.VMEM((_B, _OUT), jnp.float32),
            pltpu.VMEM((8, _OUT), jnp.float32),
        ],
    )(g, gamma.reshape(1, _OUT), beta.reshape(1, _OUT))


def kernel(x, tables, W, b, gamma, beta):
    del b
    tabT = jnp.transpose(tables, (0, 2, 1))
    W3 = W.reshape(_F, _E, _OUT)
    P = _tc_project(tabT, W3)
    x3 = x.reshape(_F, _B // _BLK, (_B // (_B // _BLK)) // 128, 128)
    g = _sc_gather(P, x3)
    return _tc_sum_bn(g, gamma, beta)
